# Initial kernel scaffold; baseline (speedup 1.0000x reference)
import jax
import jax.numpy as jnp
from jax.experimental import pallas as pl


def kernel(x, inptensor, attn_scores, W_gen, b_gen, W_cog, b_cog, out_map, inp_to_act):
    raise NotImplementedError("write your pallas kernel here")


# speedup vs baseline: 298.4027x; 298.4027x over previous
import functools

import jax
import jax.numpy as jnp
from jax import lax
from jax.experimental import pallas as pl
from jax.experimental.pallas import tpu as pltpu
from jax.experimental.pallas import tpu_sc as plsc

_NEG = -1e30


def _sc_gather_weights(W_gen, b_gen_p, out_map_p):
    V, H = W_gen.shape
    VP = out_map_p.shape[0]
    info = plsc.get_sparse_core_info()
    NC, NS, L = info.num_cores, info.num_subcores, info.num_lanes
    NW = NC * NS
    CH = 128
    n_full = V // CH
    tail = V - n_full * CH
    kmax = (n_full + NW - 1) // NW
    mesh = plsc.VectorSubcoreMesh(core_axis_name="c", subcore_axis_name="s")

    @functools.partial(
        pl.kernel,
        out_type=(jax.ShapeDtypeStruct((V, H), jnp.float32),
                  jax.ShapeDtypeStruct((VP,), jnp.float32)),
        mesh=mesh,
        compiler_params=pltpu.CompilerParams(needs_layout_passes=False),
        scratch_types=[
            pltpu.VMEM((VP,), jnp.float32),
            pltpu.VMEM((CH,), jnp.int32),
            pltpu.VMEM((CH, H), jnp.float32),
            pltpu.VMEM((CH,), jnp.float32),
            pltpu.SemaphoreType.DMA,
        ],
    )
    def k(W_hbm, b_hbm, map_hbm, W2_hbm, b2_hbm, btab, idx_v, w_v, bv_v, sem1):
        wid = lax.axis_index("s") * NC + lax.axis_index("c")
        pltpu.sync_copy(b_hbm, btab)

        def gather_b(n):
            for kk in range(n // L):
                i16 = idx_v[pl.ds(kk * L, L)]
                bv_v[pl.ds(kk * L, L)] = plsc.load_gather(btab, [i16])

        def do_chunk(base):
            pltpu.sync_copy(map_hbm.at[pl.ds(base, CH)], idx_v)
            cw = pltpu.async_copy(W_hbm.at[idx_v], w_v, sem1)
            gather_b(CH)
            cw.wait()
            pltpu.sync_copy(w_v, W2_hbm.at[pl.ds(base, CH)])
            pltpu.sync_copy(bv_v, b2_hbm.at[pl.ds(base, CH)])

        def body(kk, _):
            c = wid + kk * NW

            @pl.when(c < n_full)
            def _():
                do_chunk(c * CH)

            return 0

        lax.fori_loop(0, kmax, body, 0)

        if tail:
            @pl.when(wid == NW - 1)
            def _():
                base = n_full * CH
                pltpu.sync_copy(map_hbm.at[pl.ds(base, tail)],
                                idx_v.at[pl.ds(0, tail)])
                cw = pltpu.async_copy(W_hbm.at[idx_v.at[pl.ds(0, tail)]],
                                      w_v.at[pl.ds(0, tail)], sem1)
                gather_b(tail)
                cw.wait()
                pltpu.sync_copy(w_v.at[pl.ds(0, tail)],
                                W2_hbm.at[pl.ds(base, tail)])
                pltpu.sync_copy(bv_v.at[pl.ds(0, tail)],
                                b2_hbm.at[pl.ds(base, tail)])

    return k(W_gen, b_gen_p, out_map_p)


def _sc_ptr_scatter(inp_flat, ap_flat, ia_p, Bn, S, VP):
    info = plsc.get_sparse_core_info()
    NC, NS, L = info.num_cores, info.num_subcores, info.num_lanes
    NW = NC * NS
    RPW = Bn // NW
    SP = ((S + L - 1) // L + 3) // 4 * 4 * L
    NCH = SP // L
    TPV = VP // 128
    mesh = plsc.VectorSubcoreMesh(core_axis_name="c", subcore_axis_name="s")

    @functools.partial(
        pl.kernel,
        out_type=jax.ShapeDtypeStruct((Bn, TPV, 128), jnp.float32),
        mesh=mesh,
        compiler_params=pltpu.CompilerParams(needs_layout_passes=False),
        scratch_types=[
            pltpu.VMEM((TPV, 128), jnp.float32),
            pltpu.VMEM((SP,), jnp.int32),
            pltpu.VMEM((SP,), jnp.int32),
            pltpu.VMEM((SP,), jnp.float32),
            pltpu.SemaphoreType.DMA,
            pltpu.SemaphoreType.DMA,
        ],
    )
    def k(inp_hbm, ap_hbm, map_hbm, pv_hbm, acc, ids, act, prb, sem1, sem2):
        wid = lax.axis_index("s") * NC + lax.axis_index("c")
        zero16 = jnp.zeros((L,), jnp.float32)

        def zbody(i, _):
            for j in range(8):
                acc[i, pl.ds(j * L, L)] = zero16
            return 0

        lax.fori_loop(0, TPV, zbody, 0)

        for j in range(S // L, NCH):
            prb[pl.ds(j * L, L)] = zero16
            ids[pl.ds(j * L, L)] = jnp.zeros((L,), jnp.int32)

        def row_body(i, _):
            r = wid * RPW + i
            pltpu.sync_copy(inp_hbm.at[pl.ds(r * S, S)], ids.at[pl.ds(0, S)])
            pltpu.sync_copy(ap_hbm.at[pl.ds(r * S, S)], prb.at[pl.ds(0, S)])
            g1 = pltpu.async_copy(map_hbm.at[ids.at[pl.ds(0, 128)]],
                                  act.at[pl.ds(0, 128)], sem1)
            g2 = pltpu.async_copy(map_hbm.at[ids.at[pl.ds(128, 128)]],
                                  act.at[pl.ds(128, 128)], sem2)
            g1.wait()
            g2.wait()

            def sbody(j, _):
                a = act[pl.ds(j * L, L)]
                hi = lax.shift_right_logical(a, 7)
                lo = lax.bitwise_and(a, 127)
                plsc.addupdate_scatter(acc, [hi, lo], prb[pl.ds(j * L, L)])
                return 0

            lax.fori_loop(0, NCH, sbody, 0)
            pltpu.sync_copy(acc, pv_hbm.at[r])

            def rbody(j, _):
                a = act[pl.ds(j * L, L)]
                hi = lax.shift_right_logical(a, 7)
                lo = lax.bitwise_and(a, 127)
                plsc.store_scatter(acc, [hi, lo], zero16)
                return 0

            lax.fori_loop(0, NCH, rbody, 0)
            return 0

        lax.fori_loop(0, RPW, row_body, 0)

    return k(inp_flat, ap_flat, ia_p)


def _tc_pass1(x, W2, b2r, W_cog, b_cog2, attn_scores, TJ=2048):
    B, H = x.shape
    V = W2.shape[0]
    S = attn_scores.shape[1]
    G = pl.cdiv(V, TJ)

    def body(x_ref, w_ref, b_ref, wc_ref, bc_ref, at_ref,
             logz_ref, cog_ref, ap_ref, m_ref, s_ref):
        j = pl.program_id(0)

        @pl.when(j == 0)
        def _():
            m_ref[...] = jnp.full_like(m_ref, _NEG)
            s_ref[...] = jnp.zeros_like(s_ref)
            xf = x_ref[...]
            bc = bc_ref[...]
            a0 = jnp.sum(xf * wc_ref[0:1, :], axis=1, keepdims=True) + bc[0:1, 0:1]
            a1 = jnp.sum(xf * wc_ref[1:2, :], axis=1, keepdims=True) + bc[0:1, 1:2]
            mm = jnp.maximum(a0, a1)
            lse = mm + jnp.log(jnp.exp(a0 - mm) + jnp.exp(a1 - mm))
            cog_ref[...] = jnp.concatenate([a0 - lse, a1 - lse], axis=1)
            att = at_ref[...]
            am = jnp.max(att, axis=1, keepdims=True)
            ae = jnp.exp(att - am)
            ap_ref[...] = ae / jnp.sum(ae, axis=1, keepdims=True)

        xb = x_ref[...].astype(jnp.bfloat16)
        wb = w_ref[...].astype(jnp.bfloat16)
        logits = lax.dot_general(xb, wb, (((1,), (1,)), ((), ())),
                                 preferred_element_type=jnp.float32)
        logits = logits + b_ref[...]
        col = j * TJ + lax.broadcasted_iota(jnp.int32, logits.shape, 1)
        logits = jnp.where(col < V, logits, _NEG)
        mt = jnp.max(logits, axis=1, keepdims=True)
        m_new = jnp.maximum(m_ref[...], mt)
        s_ref[...] = (s_ref[...] * jnp.exp(m_ref[...] - m_new)
                      + jnp.sum(jnp.exp(logits - m_new), axis=1, keepdims=True))
        m_ref[...] = m_new

        @pl.when(j == G - 1)
        def _():
            logz_ref[...] = m_ref[...] + jnp.log(s_ref[...])

    return pl.pallas_call(
        body,
        grid=(G,),
        in_specs=[
            pl.BlockSpec((B, H), lambda j: (0, 0)),
            pl.BlockSpec((TJ, H), lambda j: (j, 0)),
            pl.BlockSpec((1, TJ), lambda j: (0, j)),
            pl.BlockSpec((2, H), lambda j: (0, 0)),
            pl.BlockSpec((1, 2), lambda j: (0, 0)),
            pl.BlockSpec((B, S), lambda j: (0, 0)),
        ],
        out_specs=[
            pl.BlockSpec((B, 1), lambda j: (0, 0)),
            pl.BlockSpec((B, 2), lambda j: (0, 0)),
            pl.BlockSpec((B, S), lambda j: (0, 0)),
        ],
        out_shape=[
            jax.ShapeDtypeStruct((B, 1), jnp.float32),
            jax.ShapeDtypeStruct((B, 2), jnp.float32),
            jax.ShapeDtypeStruct((B, S), jnp.float32),
        ],
        scratch_shapes=[
            pltpu.VMEM((B, 1), jnp.float32),
            pltpu.VMEM((B, 1), jnp.float32),
        ],
    )(x, W2, b2r, W_cog, b_cog2, attn_scores)


def _tc_pass2(x, W2, b2r, pv, logZ, cog, TJ=2048):
    B, H = x.shape
    V = W2.shape[0]
    G = pl.cdiv(V, TJ)

    def body(x_ref, w_ref, b_ref, pv_ref, lz_ref, cog_ref, out_ref, gen_ref):
        xb = x_ref[...].astype(jnp.bfloat16)
        wb = w_ref[...].astype(jnp.bfloat16)
        logits = lax.dot_general(xb, wb, (((1,), (1,)), ((), ())),
                                 preferred_element_type=jnp.float32)
        logits = logits + b_ref[...]
        gen = logits - lz_ref[...]
        gen_ref[...] = gen
        cg = cog_ref[...]
        a = cg[:, 0:1] + gen
        pvt = pv_ref[...].reshape(B, TJ)
        q = cg[:, 1:2] + jnp.log(jnp.where(pvt == 0.0, 1e-30, pvt))
        mx = jnp.maximum(a, q)
        mn = jnp.minimum(a, q)
        out_ref[...] = mx + jnp.log1p(jnp.exp(mn - mx))

    return pl.pallas_call(
        body,
        grid=(G,),
        in_specs=[
            pl.BlockSpec((B, H), lambda j: (0, 0)),
            pl.BlockSpec((TJ, H), lambda j: (j, 0)),
            pl.BlockSpec((1, TJ), lambda j: (0, j)),
            pl.BlockSpec((B, TJ // 128, 128), lambda j: (0, j, 0)),
            pl.BlockSpec((B, 1), lambda j: (0, 0)),
            pl.BlockSpec((B, 2), lambda j: (0, 0)),
        ],
        out_specs=[
            pl.BlockSpec((B, TJ), lambda j: (0, j)),
            pl.BlockSpec((B, TJ), lambda j: (0, j)),
        ],
        out_shape=[
            jax.ShapeDtypeStruct((B, V), jnp.float32),
            jax.ShapeDtypeStruct((B, V), jnp.float32),
        ],
    )(x, W2, b2r, pv, logZ, cog)


def kernel(x, inptensor, attn_scores, W_gen, b_gen, W_cog, b_cog,
           out_map, inp_to_act):
    B, H = x.shape
    S = inptensor.shape[1]
    V = W_gen.shape[0]
    TJ = 2048
    VP = ((V + TJ - 1) // TJ) * TJ
    out_map_p = jnp.pad(out_map, (0, VP - V))
    b_gen_p = jnp.pad(b_gen, (0, VP - V))
    ia_p = jnp.pad(inp_to_act, (0, VP - V))
    W2, b2 = _sc_gather_weights(W_gen, b_gen_p, out_map_p)
    b2r = b2.reshape(1, VP)
    logZ, cog, attn_probs = _tc_pass1(x, W2, b2r, W_cog,
                                      b_cog.reshape(1, 2), attn_scores, TJ=TJ)
    pv = _sc_ptr_scatter(inptensor.reshape(B * S), attn_probs.reshape(B * S),
                         ia_p, B, S, VP)
    out_probs, gen_probs = _tc_pass2(x, W2, b2r, pv, logZ, cog, TJ=TJ)
    return (out_probs, cog, gen_probs, attn_probs)


# speedup vs baseline: 323.2469x; 1.0833x over previous
import functools

import jax
import jax.numpy as jnp
from jax import lax
from jax.experimental import pallas as pl
from jax.experimental.pallas import tpu as pltpu
from jax.experimental.pallas import tpu_sc as plsc

_NEG = -1e30


def _sc_gather_weights(W_gen, b_gen_p, out_map_p):
    V, H = W_gen.shape
    VP = out_map_p.shape[0]
    info = plsc.get_sparse_core_info()
    NC, NS, L = info.num_cores, info.num_subcores, info.num_lanes
    NW = NC * NS
    CH = 128
    n_full = V // CH
    tail = V - n_full * CH
    kmax = (n_full + NW - 1) // NW
    mesh = plsc.VectorSubcoreMesh(core_axis_name="c", subcore_axis_name="s")

    @functools.partial(
        pl.kernel,
        out_type=(jax.ShapeDtypeStruct((V, H), jnp.float32),
                  jax.ShapeDtypeStruct((VP,), jnp.float32)),
        mesh=mesh,
        compiler_params=pltpu.CompilerParams(needs_layout_passes=False),
        scratch_types=[
            pltpu.VMEM((VP,), jnp.float32),
            pltpu.VMEM((CH,), jnp.int32),
            pltpu.VMEM((CH, H), jnp.float32),
            pltpu.VMEM((CH,), jnp.float32),
            pltpu.SemaphoreType.DMA,
        ],
    )
    def k(W_hbm, b_hbm, map_hbm, W2_hbm, b2_hbm, btab, idx_v, w_v, bv_v, sem1):
        wid = lax.axis_index("s") * NC + lax.axis_index("c")
        pltpu.sync_copy(b_hbm, btab)

        def gather_b(n):
            for kk in range(n // L):
                i16 = idx_v[pl.ds(kk * L, L)]
                bv_v[pl.ds(kk * L, L)] = plsc.load_gather(btab, [i16])

        def do_chunk(base):
            pltpu.sync_copy(map_hbm.at[pl.ds(base, CH)], idx_v)
            cw = pltpu.async_copy(W_hbm.at[idx_v], w_v, sem1)
            gather_b(CH)
            cw.wait()
            pltpu.sync_copy(w_v, W2_hbm.at[pl.ds(base, CH)])
            pltpu.sync_copy(bv_v, b2_hbm.at[pl.ds(base, CH)])

        def body(kk, _):
            c = wid + kk * NW

            @pl.when(c < n_full)
            def _():
                do_chunk(c * CH)

            return 0

        lax.fori_loop(0, kmax, body, 0)

        if tail:
            @pl.when(wid == NW - 1)
            def _():
                base = n_full * CH
                pltpu.sync_copy(map_hbm.at[pl.ds(base, tail)],
                                idx_v.at[pl.ds(0, tail)])
                cw = pltpu.async_copy(W_hbm.at[idx_v.at[pl.ds(0, tail)]],
                                      w_v.at[pl.ds(0, tail)], sem1)
                gather_b(tail)
                cw.wait()
                pltpu.sync_copy(w_v.at[pl.ds(0, tail)],
                                W2_hbm.at[pl.ds(base, tail)])
                pltpu.sync_copy(bv_v.at[pl.ds(0, tail)],
                                b2_hbm.at[pl.ds(base, tail)])

    return k(W_gen, b_gen_p, out_map_p)


def _sc_ptr_scatter(inp_flat, ap_flat, ia_p, Bn, S, VP):
    info = plsc.get_sparse_core_info()
    NC, NS, L = info.num_cores, info.num_subcores, info.num_lanes
    NW = NC * NS
    RPW = Bn // NW
    SP = ((S + L - 1) // L + 3) // 4 * 4 * L
    NCH = SP // L
    TPV = VP // 128
    mesh = plsc.VectorSubcoreMesh(core_axis_name="c", subcore_axis_name="s")

    @functools.partial(
        pl.kernel,
        out_type=jax.ShapeDtypeStruct((Bn, TPV, 128), jnp.float32),
        mesh=mesh,
        compiler_params=pltpu.CompilerParams(needs_layout_passes=False),
        scratch_types=[
            pltpu.VMEM((TPV, 128), jnp.float32),
            pltpu.VMEM((SP,), jnp.int32),
            pltpu.VMEM((SP,), jnp.int32),
            pltpu.VMEM((SP,), jnp.float32),
            pltpu.SemaphoreType.DMA,
            pltpu.SemaphoreType.DMA,
        ],
    )
    def k(inp_hbm, ap_hbm, map_hbm, pv_hbm, acc, ids, act, prb, sem1, sem2):
        wid = lax.axis_index("s") * NC + lax.axis_index("c")
        zero16 = jnp.zeros((L,), jnp.float32)

        def zbody(i, _):
            for j in range(8):
                acc[i, pl.ds(j * L, L)] = zero16
            return 0

        lax.fori_loop(0, TPV, zbody, 0)

        for j in range(S // L, NCH):
            prb[pl.ds(j * L, L)] = zero16
            ids[pl.ds(j * L, L)] = jnp.zeros((L,), jnp.int32)

        def row_body(i, _):
            r = wid * RPW + i
            pltpu.sync_copy(inp_hbm.at[pl.ds(r * S, S)], ids.at[pl.ds(0, S)])
            pltpu.sync_copy(ap_hbm.at[pl.ds(r * S, S)], prb.at[pl.ds(0, S)])
            g1 = pltpu.async_copy(map_hbm.at[ids.at[pl.ds(0, 128)]],
                                  act.at[pl.ds(0, 128)], sem1)
            g2 = pltpu.async_copy(map_hbm.at[ids.at[pl.ds(128, 128)]],
                                  act.at[pl.ds(128, 128)], sem2)
            g1.wait()
            g2.wait()

            def sbody(j, _):
                a = act[pl.ds(j * L, L)]
                hi = lax.shift_right_logical(a, 7)
                lo = lax.bitwise_and(a, 127)
                plsc.addupdate_scatter(acc, [hi, lo], prb[pl.ds(j * L, L)])
                return 0

            lax.fori_loop(0, NCH, sbody, 0)
            pltpu.sync_copy(acc, pv_hbm.at[r])

            def rbody(j, _):
                a = act[pl.ds(j * L, L)]
                hi = lax.shift_right_logical(a, 7)
                lo = lax.bitwise_and(a, 127)
                plsc.store_scatter(acc, [hi, lo], zero16)
                return 0

            lax.fori_loop(0, NCH, rbody, 0)
            return 0

        lax.fori_loop(0, RPW, row_body, 0)

    return k(inp_flat, ap_flat, ia_p)


def _tc_small(x, W_cog, b_cog2, attn_scores):
    B, H = x.shape
    S = attn_scores.shape[1]

    def body(x_ref, wc_ref, bc_ref, at_ref, cog_ref, ap_ref):
        xf = x_ref[...]
        bc = bc_ref[...]
        a0 = jnp.sum(xf * wc_ref[0:1, :], axis=1, keepdims=True) + bc[0:1, 0:1]
        a1 = jnp.sum(xf * wc_ref[1:2, :], axis=1, keepdims=True) + bc[0:1, 1:2]
        mm = jnp.maximum(a0, a1)
        lse = mm + jnp.log(jnp.exp(a0 - mm) + jnp.exp(a1 - mm))
        cog_ref[...] = jnp.concatenate([a0 - lse, a1 - lse], axis=1)
        att = at_ref[...]
        am = jnp.max(att, axis=1, keepdims=True)
        ae = jnp.exp(att - am)
        ap_ref[...] = ae / jnp.sum(ae, axis=1, keepdims=True)

    return pl.pallas_call(
        body,
        out_shape=[
            jax.ShapeDtypeStruct((B, 2), jnp.float32),
            jax.ShapeDtypeStruct((B, S), jnp.float32),
        ],
    )(x, W_cog, b_cog2, attn_scores)


def _tc_pass1(x, W2, b2r, TJ=2048):
    B, H = x.shape
    V = W2.shape[0]
    G = pl.cdiv(V, TJ)

    def body(x_ref, w_ref, b_ref, logz_ref, m_ref, s_ref):
        j = pl.program_id(0)

        @pl.when(j == 0)
        def _():
            m_ref[...] = jnp.full_like(m_ref, _NEG)
            s_ref[...] = jnp.zeros_like(s_ref)

        xb = x_ref[...].astype(jnp.bfloat16)
        wb = w_ref[...].astype(jnp.bfloat16)
        logits = lax.dot_general(xb, wb, (((1,), (1,)), ((), ())),
                                 preferred_element_type=jnp.float32)
        logits = logits + b_ref[...]
        col = j * TJ + lax.broadcasted_iota(jnp.int32, logits.shape, 1)
        logits = jnp.where(col < V, logits, _NEG)
        mt = jnp.max(logits, axis=1, keepdims=True)
        m_new = jnp.maximum(m_ref[...], mt)
        s_ref[...] = (s_ref[...] * jnp.exp(m_ref[...] - m_new)
                      + jnp.sum(jnp.exp(logits - m_new), axis=1, keepdims=True))
        m_ref[...] = m_new

        @pl.when(j == G - 1)
        def _():
            logz_ref[...] = m_ref[...] + jnp.log(s_ref[...])

    return pl.pallas_call(
        body,
        grid=(G,),
        in_specs=[
            pl.BlockSpec((B, H), lambda j: (0, 0)),
            pl.BlockSpec((TJ, H), lambda j: (j, 0)),
            pl.BlockSpec((1, TJ), lambda j: (0, j)),
        ],
        out_specs=[
            pl.BlockSpec((B, 1), lambda j: (0, 0)),
        ],
        out_shape=[
            jax.ShapeDtypeStruct((B, 1), jnp.float32),
        ],
        scratch_shapes=[
            pltpu.VMEM((B, 1), jnp.float32),
            pltpu.VMEM((B, 1), jnp.float32),
        ],
    )(x, W2, b2r)


def _tc_pass2(x, W2, b2r, pv, logZ, cog, TJ=2048):
    B, H = x.shape
    V = W2.shape[0]
    G = pl.cdiv(V, TJ)

    def body(x_ref, w_ref, b_ref, pv_ref, lz_ref, cog_ref, out_ref, gen_ref):
        xb = x_ref[...].astype(jnp.bfloat16)
        wb = w_ref[...].astype(jnp.bfloat16)
        logits = lax.dot_general(xb, wb, (((1,), (1,)), ((), ())),
                                 preferred_element_type=jnp.float32)
        logits = logits + b_ref[...]
        gen = logits - lz_ref[...]
        gen_ref[...] = gen
        cg = cog_ref[...]
        a = cg[:, 0:1] + gen
        pvt = pv_ref[...].reshape(B, TJ)
        q = cg[:, 1:2] + jnp.log(jnp.where(pvt == 0.0, 1e-30, pvt))
        mx = jnp.maximum(a, q)
        mn = jnp.minimum(a, q)
        out_ref[...] = mx + jnp.log1p(jnp.exp(mn - mx))

    return pl.pallas_call(
        body,
        grid=(G,),
        in_specs=[
            pl.BlockSpec((B, H), lambda j: (0, 0)),
            pl.BlockSpec((TJ, H), lambda j: (j, 0)),
            pl.BlockSpec((1, TJ), lambda j: (0, j)),
            pl.BlockSpec((B, TJ // 128, 128), lambda j: (0, j, 0)),
            pl.BlockSpec((B, 1), lambda j: (0, 0)),
            pl.BlockSpec((B, 2), lambda j: (0, 0)),
        ],
        out_specs=[
            pl.BlockSpec((B, TJ), lambda j: (0, j)),
            pl.BlockSpec((B, TJ), lambda j: (0, j)),
        ],
        out_shape=[
            jax.ShapeDtypeStruct((B, V), jnp.float32),
            jax.ShapeDtypeStruct((B, V), jnp.float32),
        ],
    )(x, W2, b2r, pv, logZ, cog)


def kernel(x, inptensor, attn_scores, W_gen, b_gen, W_cog, b_cog,
           out_map, inp_to_act):
    B, H = x.shape
    S = inptensor.shape[1]
    V = W_gen.shape[0]
    TJ = 2048
    VP = ((V + TJ - 1) // TJ) * TJ
    out_map_p = jnp.pad(out_map, (0, VP - V))
    b_gen_p = jnp.pad(b_gen, (0, VP - V))
    ia_p = jnp.pad(inp_to_act, (0, VP - V))
    cog, attn_probs = _tc_small(x, W_cog, b_cog.reshape(1, 2), attn_scores)
    pv = _sc_ptr_scatter(inptensor.reshape(B * S), attn_probs.reshape(B * S),
                         ia_p, B, S, VP)
    W2, b2 = _sc_gather_weights(W_gen, b_gen_p, out_map_p)
    b2r = b2.reshape(1, VP)
    (logZ,) = _tc_pass1(x, W2, b2r, TJ=TJ)
    out_probs, gen_probs = _tc_pass2(x, W2, b2r, pv, logZ, cog, TJ=TJ)
    return (out_probs, cog, gen_probs, attn_probs)


# speedup vs baseline: 324.0609x; 1.0025x over previous
import functools

import jax
import jax.numpy as jnp
from jax import lax
from jax.experimental import pallas as pl
from jax.experimental.pallas import tpu as pltpu
from jax.experimental.pallas import tpu_sc as plsc

_NEG = -1e30


def _sc_gather_weights(W_gen, b_gen_p, out_map_p):
    V, H = W_gen.shape
    VP = out_map_p.shape[0]
    info = plsc.get_sparse_core_info()
    NC, NS, L = info.num_cores, info.num_subcores, info.num_lanes
    NW = NC * NS
    CH = 128
    n_full = V // CH
    tail = V - n_full * CH
    kmax = (n_full + NW - 1) // NW
    mesh = plsc.VectorSubcoreMesh(core_axis_name="c", subcore_axis_name="s")

    @functools.partial(
        pl.kernel,
        out_type=(jax.ShapeDtypeStruct((V, H), jnp.float32),
                  jax.ShapeDtypeStruct((VP,), jnp.float32)),
        mesh=mesh,
        compiler_params=pltpu.CompilerParams(needs_layout_passes=False),
        scratch_types=[
            pltpu.VMEM((VP,), jnp.float32),
            pltpu.VMEM((CH,), jnp.int32),
            pltpu.VMEM((CH, H), jnp.float32),
            pltpu.VMEM((CH,), jnp.float32),
            pltpu.SemaphoreType.DMA,
        ],
    )
    def k(W_hbm, b_hbm, map_hbm, W2_hbm, b2_hbm, btab, idx_v, w_v, bv_v, sem1):
        wid = lax.axis_index("s") * NC + lax.axis_index("c")
        pltpu.sync_copy(b_hbm, btab)

        def gather_b(n):
            for kk in range(n // L):
                i16 = idx_v[pl.ds(kk * L, L)]
                bv_v[pl.ds(kk * L, L)] = plsc.load_gather(btab, [i16])

        def do_chunk(base):
            pltpu.sync_copy(map_hbm.at[pl.ds(base, CH)], idx_v)
            cw = pltpu.async_copy(W_hbm.at[idx_v], w_v, sem1)
            gather_b(CH)
            cw.wait()
            pltpu.sync_copy(w_v, W2_hbm.at[pl.ds(base, CH)])
            pltpu.sync_copy(bv_v, b2_hbm.at[pl.ds(base, CH)])

        def body(kk, _):
            c = wid + kk * NW

            @pl.when(c < n_full)
            def _():
                do_chunk(c * CH)

            return 0

        lax.fori_loop(0, kmax, body, 0)

        if tail:
            @pl.when(wid == NW - 1)
            def _():
                base = n_full * CH
                pltpu.sync_copy(map_hbm.at[pl.ds(base, tail)],
                                idx_v.at[pl.ds(0, tail)])
                cw = pltpu.async_copy(W_hbm.at[idx_v.at[pl.ds(0, tail)]],
                                      w_v.at[pl.ds(0, tail)], sem1)
                gather_b(tail)
                cw.wait()
                pltpu.sync_copy(w_v.at[pl.ds(0, tail)],
                                W2_hbm.at[pl.ds(base, tail)])
                pltpu.sync_copy(bv_v.at[pl.ds(0, tail)],
                                b2_hbm.at[pl.ds(base, tail)])

    return k(W_gen, b_gen_p, out_map_p)


def _sc_ptr_scatter(inp_flat, ap_flat, ia_p, Bn, S, VP):
    info = plsc.get_sparse_core_info()
    NC, NS, L = info.num_cores, info.num_subcores, info.num_lanes
    NW = NC * NS
    RPW = Bn // NW
    SP = ((S + L - 1) // L + 3) // 4 * 4 * L
    NCH = SP // L
    TPV = VP // 128
    mesh = plsc.VectorSubcoreMesh(core_axis_name="c", subcore_axis_name="s")

    @functools.partial(
        pl.kernel,
        out_type=jax.ShapeDtypeStruct((Bn, TPV, 128), jnp.float32),
        mesh=mesh,
        compiler_params=pltpu.CompilerParams(needs_layout_passes=False),
    scratch_types=[
            pltpu.VMEM((TPV, 128), jnp.float32),
            pltpu.VMEM((SP,), jnp.int32),
            pltpu.VMEM((SP,), jnp.int32),
            pltpu.VMEM((SP,), jnp.int32),
            pltpu.VMEM((SP,), jnp.float32),
            pltpu.SemaphoreType.DMA,
            pltpu.SemaphoreType.DMA,
            pltpu.SemaphoreType.DMA,
        ],
    )
    def k(inp_hbm, ap_hbm, map_hbm, pv_hbm, acc, ids, acta, actb, prb,
          semL, semG, semC):
        wid = lax.axis_index("s") * NC + lax.axis_index("c")
        zero16 = jnp.zeros((L,), jnp.float32)

        def zbody(i, _):
            for j in range(8):
                acc[i, pl.ds(j * L, L)] = zero16
            return 0

        lax.fori_loop(0, TPV, zbody, 0)

        for j in range(S // L, NCH):
            prb[pl.ds(j * L, L)] = zero16
            ids[pl.ds(j * L, L)] = jnp.zeros((L,), jnp.int32)

        def load_row(i):
            r = wid * RPW + i
            l1 = pltpu.async_copy(inp_hbm.at[pl.ds(r * S, S)],
                                  ids.at[pl.ds(0, S)], semL)
            l2 = pltpu.async_copy(ap_hbm.at[pl.ds(r * S, S)],
                                  prb.at[pl.ds(0, S)], semL)
            l1.wait()
            l2.wait()

        def gather_into(act):
            g1 = pltpu.async_copy(map_hbm.at[ids.at[pl.ds(0, 128)]],
                                  act.at[pl.ds(0, 128)], semG)
            g2 = pltpu.async_copy(map_hbm.at[ids.at[pl.ds(128, 128)]],
                                  act.at[pl.ds(128, 128)], semG)
            g1.wait()
            g2.wait()

        def scatter_from(act):
            def sbody(j, _):
                a = act[pl.ds(j * L, L)]
                hi = lax.shift_right_logical(a, 7)
                lo = lax.bitwise_and(a, 127)
                plsc.addupdate_scatter(acc, [hi, lo], prb[pl.ds(j * L, L)])
                return 0

            lax.fori_loop(0, NCH, sbody, 0)

        def rezero_from(act):
            def rbody(j, _):
                a = act[pl.ds(j * L, L)]
                hi = lax.shift_right_logical(a, 7)
                lo = lax.bitwise_and(a, 127)
                plsc.store_scatter(acc, [hi, lo], zero16)
                return 0

            lax.fori_loop(0, NCH, rbody, 0)

        load_row(0)
        gather_into(acta)

        def row_body(i, _):
            r = wid * RPW + i
            even = lax.rem(i, 2) == 0

            @pl.when(even)
            def _():
                scatter_from(acta)

            @pl.when(jnp.logical_not(even))
            def _():
                scatter_from(actb)

            cp = pltpu.async_copy(acc, pv_hbm.at[r], semC)

            @pl.when(i + 1 < RPW)
            def _():
                load_row(i + 1)

                @pl.when(even)
                def _():
                    gather_into(actb)

                @pl.when(jnp.logical_not(even))
                def _():
                    gather_into(acta)

            cp.wait()

            @pl.when(jnp.logical_and(even, i + 1 < RPW))
            def _():
                rezero_from(acta)

            @pl.when(jnp.logical_and(jnp.logical_not(even), i + 1 < RPW))
            def _():
                rezero_from(actb)

            return 0

        lax.fori_loop(0, RPW, row_body, 0)

    return k(inp_flat, ap_flat, ia_p)


def _tc_small(x, W_cog, b_cog2, attn_scores):
    B, H = x.shape
    S = attn_scores.shape[1]

    def body(x_ref, wc_ref, bc_ref, at_ref, cog_ref, ap_ref):
        xf = x_ref[...]
        bc = bc_ref[...]
        a0 = jnp.sum(xf * wc_ref[0:1, :], axis=1, keepdims=True) + bc[0:1, 0:1]
        a1 = jnp.sum(xf * wc_ref[1:2, :], axis=1, keepdims=True) + bc[0:1, 1:2]
        mm = jnp.maximum(a0, a1)
        lse = mm + jnp.log(jnp.exp(a0 - mm) + jnp.exp(a1 - mm))
        cog_ref[...] = jnp.concatenate([a0 - lse, a1 - lse], axis=1)
        att = at_ref[...]
        am = jnp.max(att, axis=1, keepdims=True)
        ae = jnp.exp(att - am)
        ap_ref[...] = ae / jnp.sum(ae, axis=1, keepdims=True)

    return pl.pallas_call(
        body,
        out_shape=[
            jax.ShapeDtypeStruct((B, 2), jnp.float32),
            jax.ShapeDtypeStruct((B, S), jnp.float32),
        ],
    )(x, W_cog, b_cog2, attn_scores)


def _tc_pass1(x, W2, b2r, TJ=2048):
    B, H = x.shape
    V = W2.shape[0]
    G = pl.cdiv(V, TJ)

    def body(x_ref, w_ref, b_ref, logz_ref, m_ref, s_ref):
        j = pl.program_id(0)

        @pl.when(j == 0)
        def _():
            m_ref[...] = jnp.full_like(m_ref, _NEG)
            s_ref[...] = jnp.zeros_like(s_ref)

        xb = x_ref[...].astype(jnp.bfloat16)
        wb = w_ref[...].astype(jnp.bfloat16)
        logits = lax.dot_general(xb, wb, (((1,), (1,)), ((), ())),
                                 preferred_element_type=jnp.float32)
        logits = logits + b_ref[...]
        col = j * TJ + lax.broadcasted_iota(jnp.int32, logits.shape, 1)
        logits = jnp.where(col < V, logits, _NEG)
        mt = jnp.max(logits, axis=1, keepdims=True)
        m_new = jnp.maximum(m_ref[...], mt)
        s_ref[...] = (s_ref[...] * jnp.exp(m_ref[...] - m_new)
                      + jnp.sum(jnp.exp(logits - m_new), axis=1, keepdims=True))
        m_ref[...] = m_new

        @pl.when(j == G - 1)
        def _():
            logz_ref[...] = m_ref[...] + jnp.log(s_ref[...])

    return pl.pallas_call(
        body,
        grid=(G,),
        in_specs=[
            pl.BlockSpec((B, H), lambda j: (0, 0)),
            pl.BlockSpec((TJ, H), lambda j: (j, 0)),
            pl.BlockSpec((1, TJ), lambda j: (0, j)),
        ],
        out_specs=[
            pl.BlockSpec((B, 1), lambda j: (0, 0)),
        ],
        out_shape=[
            jax.ShapeDtypeStruct((B, 1), jnp.float32),
        ],
        scratch_shapes=[
            pltpu.VMEM((B, 1), jnp.float32),
            pltpu.VMEM((B, 1), jnp.float32),
        ],
    )(x, W2, b2r)


def _tc_pass2(x, W2, b2r, pv, logZ, cog, TJ=2048):
    B, H = x.shape
    V = W2.shape[0]
    G = pl.cdiv(V, TJ)

    def body(x_ref, w_ref, b_ref, pv_ref, lz_ref, cog_ref, out_ref, gen_ref):
        xb = x_ref[...].astype(jnp.bfloat16)
        wb = w_ref[...].astype(jnp.bfloat16)
        logits = lax.dot_general(xb, wb, (((1,), (1,)), ((), ())),
                                 preferred_element_type=jnp.float32)
        logits = logits + b_ref[...]
        gen = logits - lz_ref[...]
        gen_ref[...] = gen
        cg = cog_ref[...]
        a = cg[:, 0:1] + gen
        pvt = pv_ref[...].reshape(B, TJ)
        q = cg[:, 1:2] + jnp.log(jnp.where(pvt == 0.0, 1e-30, pvt))
        mx = jnp.maximum(a, q)
        mn = jnp.minimum(a, q)
        out_ref[...] = mx + jnp.log1p(jnp.exp(mn - mx))

    return pl.pallas_call(
        body,
        grid=(G,),
        in_specs=[
            pl.BlockSpec((B, H), lambda j: (0, 0)),
            pl.BlockSpec((TJ, H), lambda j: (j, 0)),
            pl.BlockSpec((1, TJ), lambda j: (0, j)),
            pl.BlockSpec((B, TJ // 128, 128), lambda j: (0, j, 0)),
            pl.BlockSpec((B, 1), lambda j: (0, 0)),
            pl.BlockSpec((B, 2), lambda j: (0, 0)),
        ],
        out_specs=[
            pl.BlockSpec((B, TJ), lambda j: (0, j)),
            pl.BlockSpec((B, TJ), lambda j: (0, j)),
        ],
        out_shape=[
            jax.ShapeDtypeStruct((B, V), jnp.float32),
            jax.ShapeDtypeStruct((B, V), jnp.float32),
        ],
    )(x, W2, b2r, pv, logZ, cog)


def kernel(x, inptensor, attn_scores, W_gen, b_gen, W_cog, b_cog,
           out_map, inp_to_act):
    B, H = x.shape
    S = inptensor.shape[1]
    V = W_gen.shape[0]
    TJ = 2048
    VP = ((V + TJ - 1) // TJ) * TJ
    out_map_p = jnp.pad(out_map, (0, VP - V))
    b_gen_p = jnp.pad(b_gen, (0, VP - V))
    ia_p = jnp.pad(inp_to_act, (0, VP - V))
    cog, attn_probs = _tc_small(x, W_cog, b_cog.reshape(1, 2), attn_scores)
    pv = _sc_ptr_scatter(inptensor.reshape(B * S), attn_probs.reshape(B * S),
                         ia_p, B, S, VP)
    W2, b2 = _sc_gather_weights(W_gen, b_gen_p, out_map_p)
    b2r = b2.reshape(1, VP)
    (logZ,) = _tc_pass1(x, W2, b2r, TJ=TJ)
    out_probs, gen_probs = _tc_pass2(x, W2, b2r, pv, logZ, cog, TJ=TJ)
    return (out_probs, cog, gen_probs, attn_probs)
